# baseline (speedup 1.0000x reference)
import functools

import jax
import jax.numpy as jnp
from jax import lax
from jax.experimental import pallas as pl
from jax.experimental.pallas import tpu as pltpu, tpu_sc as plsc

GAMMA = 24.0
DIM = 64
PKD = DIM // 2
NEG = 128
BATCH = 4096
NUM_CORES = 2
NUM_SUBCORES = 16
NW = NUM_CORES * NUM_SUBCORES
BPW = BATCH // NW
LANES = 16
NGROUPS = NEG // LANES
PGROUPS = PKD // LANES


def _unpack(w):
  lo, hi = plsc.unpack(plsc.bitcast(w, jnp.bfloat16),
                       format=plsc.PackFormat.INTERLEAVED,
                       preferred_element_type=jnp.float32)
  return lo, hi


@functools.cache
def _build():
  mesh = plsc.VectorSubcoreMesh(
      core_axis_name="c", subcore_axis_name="s",
      num_cores=NUM_CORES, num_subcores=NUM_SUBCORES)

  @functools.partial(
      pl.kernel,
      out_type=(jax.ShapeDtypeStruct((BATCH,), jnp.float32),
                jax.ShapeDtypeStruct((BATCH, NEG), jnp.float32)),
      mesh=mesh,
      compiler_params=pltpu.CompilerParams(
          needs_layout_passes=False, use_tc_tiling_on_sc=False),
      scratch_types=dict(
          qpk_v=pltpu.VMEM((BPW,), jnp.int32),
          q0_v=pltpu.VMEM((BPW,), jnp.int32),
          q1_v=pltpu.VMEM((BPW,), jnp.int32),
          pos_v=pltpu.VMEM((BPW,), jnp.int32),
          neg_v=pltpu.VMEM((BPW, NEG), jnp.int32),
          rotbuf_v=pltpu.VMEM((PKD, LANES), jnp.int32),
          clo_v=pltpu.VMEM((PKD,), jnp.float32),
          chi_v=pltpu.VMEM((PKD,), jnp.float32),
          anchor_v=pltpu.VMEM((BPW, PKD), jnp.int32),
          rel_v=pltpu.VMEM((BPW, PKD), jnp.int32),
          posrow_v=pltpu.VMEM((BPW, PKD), jnp.int32),
          nbuf0_v=pltpu.VMEM((NEG, PKD), jnp.int32),
          nbuf1_v=pltpu.VMEM((NEG, PKD), jnp.int32),
          outp_v=pltpu.VMEM((BPW,), jnp.float32),
          outn_v=pltpu.VMEM((BPW, NEG), jnp.float32),
          sem_idx=pltpu.SemaphoreType.DMA,
          sem_pre=pltpu.SemaphoreType.DMA,
          sem_n0=pltpu.SemaphoreType.DMA,
          sem_n1=pltpu.SemaphoreType.DMA,
      ),
  )
  def _gqe_sc(pos_hbm, neg_hbm, q_hbm, ent_hbm, rel_hbm, outp_hbm, outn_hbm,
              qpk_v, q0_v, q1_v, pos_v, neg_v, rotbuf_v, clo_v, chi_v,
              anchor_v, rel_v, posrow_v, nbuf0_v, nbuf1_v, outp_v, outn_v,
              sem_idx, sem_pre, sem_n0, sem_n1):
    wid = lax.axis_index("s") * NUM_CORES + lax.axis_index("c")
    base = wid * BPW

    pltpu.make_async_copy(q_hbm.at[pl.ds(base, BPW)], qpk_v, sem_idx).start()
    pltpu.make_async_copy(pos_hbm.at[pl.ds(base, BPW)], pos_v, sem_idx).start()
    pltpu.make_async_copy(neg_hbm.at[pl.ds(base, BPW)], neg_v, sem_idx).start()

    lane = lax.iota(jnp.int32, LANES)

    rot = lane & (PKD - 1)
    for d in range(PKD):
      rotbuf_v[d, pl.ds(0, LANES)] = rot
      rot = (rot + 1) & (PKD - 1)

    pltpu.make_async_copy(q_hbm.at[pl.ds(base, BPW)], qpk_v, sem_idx).wait()
    pltpu.make_async_copy(pos_hbm.at[pl.ds(base, BPW)], pos_v, sem_idx).wait()
    pltpu.make_async_copy(neg_hbm.at[pl.ds(base, BPW)], neg_v, sem_idx).wait()

    for k in range(BPW // LANES):
      sl = pl.ds(k * LANES, LANES)
      v = qpk_v[sl]
      q0_v[sl] = v & 0xFFFF
      q1_v[sl] = lax.shift_right_logical(v, 16)

    pltpu.make_async_copy(ent_hbm.at[q0_v], anchor_v, sem_pre).start()
    pltpu.make_async_copy(rel_hbm.at[q1_v], rel_v, sem_pre).start()
    pltpu.make_async_copy(ent_hbm.at[pos_v], posrow_v, sem_pre).start()

    def start_neg(row, buf, sem):
      pltpu.make_async_copy(ent_hbm.at[neg_v.at[row]], buf, sem).start()

    def wait_neg(row, buf, sem):
      pltpu.make_async_copy(ent_hbm.at[neg_v.at[row]], buf, sem).wait()

    start_neg(0, nbuf0_v, sem_n0)
    start_neg(1, nbuf1_v, sem_n1)

    pltpu.make_async_copy(ent_hbm.at[q0_v], anchor_v, sem_pre).wait()
    pltpu.make_async_copy(rel_hbm.at[q1_v], rel_v, sem_pre).wait()
    pltpu.make_async_copy(ent_hbm.at[pos_v], posrow_v, sem_pre).wait()

    row_ids = [lane + g * LANES for g in range(NGROUPS)]

    def compute_row(r, nbuf):
      for k in range(PGROUPS):
        sl = pl.ds(k * LANES, LANES)
        a_lo, a_hi = _unpack(anchor_v[r, sl])
        r_lo, r_hi = _unpack(rel_v[r, sl])
        clo_v[sl] = a_lo + r_lo
        chi_v[sl] = a_hi + r_hi
      accs = [jnp.zeros((LANES,), jnp.float32) for _ in range(NGROUPS)]
      for d in range(PKD):
        rot_d = rotbuf_v[d, pl.ds(0, LANES)]
        c_lo = plsc.load_gather(clo_v, [rot_d])
        c_hi = plsc.load_gather(chi_v, [rot_d])
        for g in range(NGROUPS):
          w = plsc.load_gather(nbuf, [row_ids[g], rot_d])
          v_lo, v_hi = _unpack(w)
          accs[g] = accs[g] + jnp.abs(v_lo - c_lo) + jnp.abs(v_hi - c_hi)
      for g in range(NGROUPS):
        outn_v[r, pl.ds(g * LANES, LANES)] = GAMMA - accs[g]

    def body(i, carry):
      r = i * 2
      wait_neg(r, nbuf0_v, sem_n0)
      compute_row(r, nbuf0_v)

      @pl.when(i < BPW // 2 - 1)
      def _():
        start_neg(r + 2, nbuf0_v, sem_n0)

      wait_neg(r + 1, nbuf1_v, sem_n1)
      compute_row(r + 1, nbuf1_v)

      @pl.when(i < BPW // 2 - 1)
      def _():
        start_neg(r + 3, nbuf1_v, sem_n1)

      return carry

    lax.fori_loop(0, BPW // 2, body, 0)

    for rg in range(NGROUPS):
      rows = lane + rg * LANES
      acc = jnp.zeros((LANES,), jnp.float32)
      for d in range(PKD):
        rot_d = rotbuf_v[d, pl.ds(0, LANES)]
        p_lo, p_hi = _unpack(plsc.load_gather(posrow_v, [rows, rot_d]))
        a_lo, a_hi = _unpack(plsc.load_gather(anchor_v, [rows, rot_d]))
        r_lo, r_hi = _unpack(plsc.load_gather(rel_v, [rows, rot_d]))
        acc = (acc + jnp.abs(p_lo - a_lo - r_lo)
               + jnp.abs(p_hi - a_hi - r_hi))
      outp_v[pl.ds(rg * LANES, LANES)] = GAMMA - acc

    pltpu.sync_copy(outp_v, outp_hbm.at[pl.ds(base, BPW)])
    pltpu.sync_copy(outn_v, outn_hbm.at[pl.ds(base, BPW)])

  return _gqe_sc


def _pack_table(t):
  bf = t.astype(jnp.bfloat16)
  return jax.lax.bitcast_convert_type(
      bf.reshape(t.shape[0], t.shape[1] // 2, 2), jnp.int32)


def kernel(positive_sample, negative_sample, subsampling_weight, queries,
           entity_embedding, relation_embedding):
  del subsampling_weight
  qpacked = queries[:, 0] + (queries[:, 1] << 16)
  pos_logit, neg_logit = _build()(
      positive_sample, negative_sample, qpacked,
      _pack_table(entity_embedding), _pack_table(relation_embedding))
  return jnp.concatenate([pos_logit[:, None], neg_logit], axis=1)


# speedup vs baseline: 1.8260x; 1.8260x over previous
import functools

import jax
import jax.numpy as jnp
from jax import lax
from jax.experimental import pallas as pl
from jax.experimental.pallas import tpu as pltpu, tpu_sc as plsc

GAMMA = 24.0
DIM = 64
NEG = 128
BATCH = 4096
NUM_CORES = 2
NUM_SUBCORES = 16
NW = NUM_CORES * NUM_SUBCORES
BPW = BATCH // NW
LANES = 16
NGROUPS = NEG // LANES
DGROUPS = DIM // LANES


@functools.cache
def _build():
  mesh = plsc.VectorSubcoreMesh(
      core_axis_name="c", subcore_axis_name="s",
      num_cores=NUM_CORES, num_subcores=NUM_SUBCORES)

  @functools.partial(
      pl.kernel,
      out_type=(jax.ShapeDtypeStruct((BATCH,), jnp.float32),
                jax.ShapeDtypeStruct((BATCH, NEG), jnp.float32)),
      mesh=mesh,
      compiler_params=pltpu.CompilerParams(
          needs_layout_passes=False, use_tc_tiling_on_sc=False),
      scratch_types=dict(
          qpk_v=pltpu.VMEM((BPW,), jnp.int32),
          q0_v=pltpu.VMEM((BPW,), jnp.int32),
          q1_v=pltpu.VMEM((BPW,), jnp.int32),
          pos_v=pltpu.VMEM((BPW,), jnp.int32),
          neg_v=pltpu.VMEM((BPW, NEG), jnp.int32),
          rotbuf_v=pltpu.VMEM((DIM, LANES), jnp.int32),
          cbuf_v=pltpu.VMEM((DIM,), jnp.float32),
          anchor_v=pltpu.VMEM((BPW, DIM), jnp.float32),
          rel_v=pltpu.VMEM((BPW, DIM), jnp.float32),
          posrow_v=pltpu.VMEM((BPW, DIM), jnp.float32),
          nbuf0_v=pltpu.VMEM((NEG, DIM), jnp.float32),
          nbuf1_v=pltpu.VMEM((NEG, DIM), jnp.float32),
          outp_v=pltpu.VMEM((BPW,), jnp.float32),
          outn_v=pltpu.VMEM((BPW, NEG), jnp.float32),
          sem_idx=pltpu.SemaphoreType.DMA,
          sem_pre=pltpu.SemaphoreType.DMA,
          sem_n0=pltpu.SemaphoreType.DMA,
          sem_n1=pltpu.SemaphoreType.DMA,
      ),
  )
  def _gqe_sc(pos_hbm, neg_hbm, q_hbm, ent_hbm, rel_hbm, outp_hbm, outn_hbm,
              qpk_v, q0_v, q1_v, pos_v, neg_v, rotbuf_v, cbuf_v,
              anchor_v, rel_v, posrow_v, nbuf0_v, nbuf1_v, outp_v, outn_v,
              sem_idx, sem_pre, sem_n0, sem_n1):
    wid = lax.axis_index("s") * NUM_CORES + lax.axis_index("c")
    base = wid * BPW

    pltpu.make_async_copy(q_hbm.at[pl.ds(base, BPW)], qpk_v, sem_idx).start()
    pltpu.make_async_copy(pos_hbm.at[pl.ds(base, BPW)], pos_v, sem_idx).start()
    pltpu.make_async_copy(neg_hbm.at[pl.ds(base, BPW)], neg_v, sem_idx).start()

    lane = lax.iota(jnp.int32, LANES)

    rot = lane
    for d in range(DIM):
      rotbuf_v[d, pl.ds(0, LANES)] = rot
      rot = (rot + 1) & (DIM - 1)

    pltpu.make_async_copy(q_hbm.at[pl.ds(base, BPW)], qpk_v, sem_idx).wait()
    pltpu.make_async_copy(pos_hbm.at[pl.ds(base, BPW)], pos_v, sem_idx).wait()
    pltpu.make_async_copy(neg_hbm.at[pl.ds(base, BPW)], neg_v, sem_idx).wait()

    zcol = jnp.zeros((LANES,), jnp.int32)
    for k in range(BPW // LANES):
      sl = pl.ds(k * LANES, LANES)
      v = qpk_v[sl]
      q0_v[sl] = v & 0xFFFF
      q1_v[sl] = lax.shift_right_logical(v, 16)

    pltpu.make_async_copy(ent_hbm.at[q0_v], anchor_v, sem_pre).start()
    pltpu.make_async_copy(rel_hbm.at[q1_v], rel_v, sem_pre).start()
    pltpu.make_async_copy(ent_hbm.at[pos_v], posrow_v, sem_pre).start()

    H = NEG // 2

    def start_neg(row, buf, sem):
      pltpu.make_async_copy(
          ent_hbm.at[neg_v.at[row, pl.ds(0, H)]],
          buf.at[pl.ds(0, H)], sem).start()
      pltpu.make_async_copy(
          ent_hbm.at[neg_v.at[row, pl.ds(H, H)]],
          buf.at[pl.ds(H, H)], sem).start()

    def wait_neg(row, buf, sem):
      pltpu.make_async_copy(
          ent_hbm.at[neg_v.at[row, pl.ds(0, H)]],
          buf.at[pl.ds(0, H)], sem).wait()
      pltpu.make_async_copy(
          ent_hbm.at[neg_v.at[row, pl.ds(H, H)]],
          buf.at[pl.ds(H, H)], sem).wait()

    start_neg(0, nbuf0_v, sem_n0)
    start_neg(1, nbuf1_v, sem_n1)

    pltpu.make_async_copy(ent_hbm.at[q0_v], anchor_v, sem_pre).wait()
    pltpu.make_async_copy(rel_hbm.at[q1_v], rel_v, sem_pre).wait()
    pltpu.make_async_copy(ent_hbm.at[pos_v], posrow_v, sem_pre).wait()

    row_ids = [lane + g * LANES for g in range(NGROUPS)]

    def compute_row(r, nbuf):
      for k in range(DGROUPS):
        sl = pl.ds(k * LANES, LANES)
        cbuf_v[sl] = anchor_v[r, sl] + rel_v[r, sl]
      accs = [jnp.zeros((LANES,), jnp.float32) for _ in range(NGROUPS)]
      for d in range(DIM):
        rot_d = rotbuf_v[d, pl.ds(0, LANES)]
        c = plsc.load_gather(cbuf_v, [rot_d])
        for g in range(NGROUPS):
          vals = plsc.load_gather(nbuf, [row_ids[g], rot_d])
          accs[g] = accs[g] + jnp.abs(vals - c)
      for g in range(NGROUPS):
        outn_v[r, pl.ds(g * LANES, LANES)] = GAMMA - accs[g]

    def body(i, carry):
      r = i * 2
      wait_neg(r, nbuf0_v, sem_n0)
      compute_row(r, nbuf0_v)

      @pl.when(i < BPW // 2 - 1)
      def _():
        start_neg(r + 2, nbuf0_v, sem_n0)

      wait_neg(r + 1, nbuf1_v, sem_n1)
      compute_row(r + 1, nbuf1_v)

      @pl.when(i < BPW // 2 - 1)
      def _():
        start_neg(r + 3, nbuf1_v, sem_n1)

      return carry

    lax.fori_loop(0, BPW // 2, body, 0)

    for rg in range(NGROUPS):
      rows = lane + rg * LANES
      acc = jnp.zeros((LANES,), jnp.float32)
      for d in range(DIM):
        rot_d = rotbuf_v[d, pl.ds(0, LANES)]
        pvals = plsc.load_gather(posrow_v, [rows, rot_d])
        avals = plsc.load_gather(anchor_v, [rows, rot_d])
        rvals = plsc.load_gather(rel_v, [rows, rot_d])
        acc = acc + jnp.abs(pvals - avals - rvals)
      outp_v[pl.ds(rg * LANES, LANES)] = GAMMA - acc

    pltpu.sync_copy(outp_v, outp_hbm.at[pl.ds(base, BPW)])
    pltpu.sync_copy(outn_v, outn_hbm.at[pl.ds(base, BPW)])

  return _gqe_sc


def kernel(positive_sample, negative_sample, subsampling_weight, queries,
           entity_embedding, relation_embedding):
  del subsampling_weight
  qpacked = queries[:, 0] + (queries[:, 1] << 16)
  pos_logit, neg_logit = _build()(positive_sample, negative_sample, qpacked,
                                  entity_embedding, relation_embedding)
  return jnp.concatenate([pos_logit[:, None], neg_logit], axis=1)


# speedup vs baseline: 2.2111x; 1.2109x over previous
import functools

import jax
import jax.numpy as jnp
from jax import lax
from jax.experimental import pallas as pl
from jax.experimental.pallas import tpu as pltpu, tpu_sc as plsc

GAMMA = 24.0
DIM = 64
NEG = 128
BATCH = 4096
NUM_CORES = 2
NUM_SUBCORES = 16
NW = NUM_CORES * NUM_SUBCORES
BPW = BATCH // NW
LANES = 16
NGROUPS = NEG // LANES
DGROUPS = DIM // LANES


@functools.cache
def _build():
  mesh = plsc.VectorSubcoreMesh(
      core_axis_name="c", subcore_axis_name="s",
      num_cores=NUM_CORES, num_subcores=NUM_SUBCORES)

  @functools.partial(
      pl.kernel,
      out_type=(jax.ShapeDtypeStruct((BATCH,), jnp.float32),
                jax.ShapeDtypeStruct((BATCH, NEG), jnp.float32)),
      mesh=mesh,
      compiler_params=pltpu.CompilerParams(
          needs_layout_passes=False, use_tc_tiling_on_sc=False),
      scratch_types=dict(
          qpk_v=pltpu.VMEM((BPW,), jnp.int32),
          q0_v=pltpu.VMEM((BPW,), jnp.int32),
          q1_v=pltpu.VMEM((BPW,), jnp.int32),
          pos_v=pltpu.VMEM((BPW,), jnp.int32),
          neg_v=pltpu.VMEM((BPW, NEG), jnp.int32),
          rotbuf_v=pltpu.VMEM((DIM, LANES), jnp.int32),
          cbuf_v=pltpu.VMEM((DIM,), jnp.float32),
          anchor_v=pltpu.VMEM((BPW, DIM), jnp.float32),
          rel_v=pltpu.VMEM((BPW, DIM), jnp.float32),
          posrow_v=pltpu.VMEM((BPW, DIM), jnp.float32),
          nbuf0_v=pltpu.VMEM((NEG, DIM), jnp.float32),
          nbuf1_v=pltpu.VMEM((NEG, DIM), jnp.float32),
          outp_v=pltpu.VMEM((BPW,), jnp.float32),
          outn_v=pltpu.VMEM((BPW, NEG), jnp.float32),
          sem_idx=pltpu.SemaphoreType.DMA,
          sem_pre=pltpu.SemaphoreType.DMA,
          sem_n0=pltpu.SemaphoreType.DMA,
          sem_n1=pltpu.SemaphoreType.DMA,
      ),
  )
  def _gqe_sc(pos_hbm, neg_hbm, q_hbm, ent_hbm, rel_hbm, outp_hbm, outn_hbm,
              qpk_v, q0_v, q1_v, pos_v, neg_v, rotbuf_v, cbuf_v,
              anchor_v, rel_v, posrow_v, nbuf0_v, nbuf1_v, outp_v, outn_v,
              sem_idx, sem_pre, sem_n0, sem_n1):
    wid = lax.axis_index("s") * NUM_CORES + lax.axis_index("c")
    base = wid * BPW

    pltpu.make_async_copy(q_hbm.at[pl.ds(base, BPW)], qpk_v, sem_idx).start()
    pltpu.make_async_copy(pos_hbm.at[pl.ds(base, BPW)], pos_v, sem_idx).start()
    pltpu.make_async_copy(neg_hbm.at[pl.ds(base, BPW)], neg_v, sem_idx).start()

    lane = lax.iota(jnp.int32, LANES)

    rot = lane
    for d in range(DIM):
      rotbuf_v[d, pl.ds(0, LANES)] = rot
      rot = (rot + 1) & (DIM - 1)

    pltpu.make_async_copy(q_hbm.at[pl.ds(base, BPW)], qpk_v, sem_idx).wait()
    pltpu.make_async_copy(pos_hbm.at[pl.ds(base, BPW)], pos_v, sem_idx).wait()
    pltpu.make_async_copy(neg_hbm.at[pl.ds(base, BPW)], neg_v, sem_idx).wait()

    zcol = jnp.zeros((LANES,), jnp.int32)
    for k in range(BPW // LANES):
      sl = pl.ds(k * LANES, LANES)
      v = qpk_v[sl]
      q0_v[sl] = v & 0xFFFF
      q1_v[sl] = lax.shift_right_logical(v, 16)

    pltpu.make_async_copy(ent_hbm.at[q0_v], anchor_v, sem_pre).start()
    pltpu.make_async_copy(rel_hbm.at[q1_v], rel_v, sem_pre).start()
    pltpu.make_async_copy(ent_hbm.at[pos_v], posrow_v, sem_pre).start()

    def start_neg(row, buf, sem):
      pltpu.make_async_copy(ent_hbm.at[neg_v.at[row]], buf, sem).start()

    def wait_neg(row, buf, sem):
      pltpu.make_async_copy(ent_hbm.at[neg_v.at[row]], buf, sem).wait()

    start_neg(0, nbuf0_v, sem_n0)
    start_neg(1, nbuf1_v, sem_n1)

    pltpu.make_async_copy(ent_hbm.at[q0_v], anchor_v, sem_pre).wait()
    pltpu.make_async_copy(rel_hbm.at[q1_v], rel_v, sem_pre).wait()
    pltpu.make_async_copy(ent_hbm.at[pos_v], posrow_v, sem_pre).wait()

    row_ids = [lane + g * LANES for g in range(NGROUPS)]

    def compute_row(r, nbuf):
      for k in range(DGROUPS):
        sl = pl.ds(k * LANES, LANES)
        cbuf_v[sl] = anchor_v[r, sl] + rel_v[r, sl]
      accs = [jnp.zeros((LANES,), jnp.float32) for _ in range(NGROUPS)]
      for d in range(DIM):
        rot_d = rotbuf_v[d, pl.ds(0, LANES)]
        c = plsc.load_gather(cbuf_v, [rot_d])
        for g in range(NGROUPS):
          vals = plsc.load_gather(nbuf, [row_ids[g], rot_d])
          accs[g] = accs[g] + jnp.abs(vals - c)
      for g in range(NGROUPS):
        outn_v[r, pl.ds(g * LANES, LANES)] = GAMMA - accs[g]

    def body(i, carry):
      r = i * 2
      wait_neg(r, nbuf0_v, sem_n0)
      compute_row(r, nbuf0_v)

      @pl.when(i < BPW // 2 - 1)
      def _():
        start_neg(r + 2, nbuf0_v, sem_n0)

      wait_neg(r + 1, nbuf1_v, sem_n1)
      compute_row(r + 1, nbuf1_v)

      @pl.when(i < BPW // 2 - 1)
      def _():
        start_neg(r + 3, nbuf1_v, sem_n1)

      return carry

    lax.fori_loop(0, BPW // 2, body, 0)

    for rg in range(NGROUPS):
      rows = lane + rg * LANES
      acc = jnp.zeros((LANES,), jnp.float32)
      for d in range(DIM):
        rot_d = rotbuf_v[d, pl.ds(0, LANES)]
        pvals = plsc.load_gather(posrow_v, [rows, rot_d])
        avals = plsc.load_gather(anchor_v, [rows, rot_d])
        rvals = plsc.load_gather(rel_v, [rows, rot_d])
        acc = acc + jnp.abs(pvals - avals - rvals)
      outp_v[pl.ds(rg * LANES, LANES)] = GAMMA - acc

    pltpu.sync_copy(outp_v, outp_hbm.at[pl.ds(base, BPW)])
    pltpu.sync_copy(outn_v, outn_hbm.at[pl.ds(base, BPW)])

  return _gqe_sc


def kernel(positive_sample, negative_sample, subsampling_weight, queries,
           entity_embedding, relation_embedding):
  del subsampling_weight
  qpacked = queries[:, 0] + (queries[:, 1] << 16)
  pos_logit, neg_logit = _build()(positive_sample, negative_sample, qpacked,
                                  entity_embedding, relation_embedding)
  return jnp.concatenate([pos_logit[:, None], neg_logit], axis=1)
